# Initial kernel scaffold; baseline (speedup 1.0000x reference)
import jax
import jax.numpy as jnp
from jax.experimental import pallas as pl


def kernel(x, edge_index, edge_attr, W_lin, att_src, att_dst, W_edge, att_edge, bias_gat, W_ih0, W_hh0, b_ih0, b_hh0, W_ih1, W_hh1, b_ih1, b_hh1, W_ih2, W_hh2, b_ih2, b_hh2, ln_gamma, ln_beta):
    raise NotImplementedError("write your pallas kernel here")


# speedup vs baseline: 31.8664x; 31.8664x over previous
import functools

import jax
import jax.numpy as jnp
import numpy as np
from jax import lax
from jax.experimental import pallas as pl
from jax.experimental.pallas import tpu as pltpu
from jax.experimental.pallas import tpu_sc as plsc

N = 10000
E = 320000
C = 128
H = 4
CO = 32
HID = 128
ED = 16

NC = 2
NS = 16
NW = NC * NS
EPW = E // NW
K = 80
NCHUNK = EPW // K
NP = 10240
RPT = NP // NS

NBLK = 2000
NGRID = N // NBLK
EBLK = 512
EGRID = E // EBLK


def _tc_proj_node_body(x_ref, wlin_ref, asd_ref, xt_ref, sd_ref):
    x = x_ref[...]
    xt = lax.dot_general(x, wlin_ref[...], (((1,), (1,)), ((), ())),
                         preferred_element_type=jnp.float32)
    xt_ref[...] = xt
    sd_ref[...] = lax.dot_general(xt, asd_ref[...], (((1,), (0,)), ((), ())),
                                  preferred_element_type=jnp.float32)


def _tc_proj_edge_body(ea_ref, wedge_ref, ae8_ref, ae_ref):
    a_e8 = lax.dot_general(wedge_ref[...], ae8_ref[...], (((0,), (0,)), ((), ())),
                           preferred_element_type=jnp.float32)
    ae_ref[...] = lax.dot_general(ea_ref[...], a_e8, (((1,), (0,)), ((), ())),
                                  preferred_element_type=jnp.float32)


def _sc_edge_body(xt_hbm, sd_hbm, ae_hbm, src_hbm, dst_hbm, z128_hbm, z8_hbm,
                  acc_out, asum_out,
                  src_v, dst_v, ae_v, srows, drows, xrows, p_buf,
                  sd_sh, acc_sh, asum_sh, sem_x, sem_s, sem_d):
    cid = lax.axis_index("c")
    sid = lax.axis_index("s")
    wid = cid * NS + sid
    spt = N // NS

    pltpu.sync_copy(sd_hbm.at[pl.ds(sid * spt, spt)],
                    sd_sh.at[pl.ds(sid * spt, spt)])
    pltpu.sync_copy(z128_hbm.at[pl.ds(sid * RPT, RPT)],
                    acc_sh.at[pl.ds(sid * RPT, RPT)])
    pltpu.sync_copy(z8_hbm.at[pl.ds(sid * RPT, RPT)],
                    asum_sh.at[pl.ds(sid * RPT, RPT)])
    plsc.subcore_barrier()

    iota16 = lax.iota(jnp.int32, 16)

    def chunk(i, carry):
        base = wid * EPW + i * K
        pltpu.sync_copy(src_hbm.at[pl.ds(base, K)], src_v)
        pltpu.sync_copy(dst_hbm.at[pl.ds(base, K)], dst_v)
        pltpu.sync_copy(ae_hbm.at[pl.ds(base, K)], ae_v)
        cx = pltpu.async_copy(xt_hbm.at[src_v], xrows, sem_x)
        cs = pltpu.async_copy(sd_sh.at[src_v], srows, sem_s)
        cd = pltpu.async_copy(sd_sh.at[dst_v], drows, sem_d)
        cs.wait()
        cd.wait()

        for k16 in range(K // 16):
            rows = iota16 + (k16 * 16)
            for h in range(H):
                hv = jnp.full((16,), h, jnp.int32)
                hv4 = jnp.full((16,), h + 4, jnp.int32)
                sv = plsc.load_gather(srows, [rows, hv])
                dv = plsc.load_gather(drows, [rows, hv4])
                av = plsc.load_gather(ae_v, [rows, hv])
                al = sv + dv + av
                al = jnp.where(al > 0.0, al, al * 0.2)
                plsc.store_scatter(p_buf, [rows, hv], jnp.exp(al))

        cx.wait()

        def scale(k, c2):
            kv = jnp.full((16,), k, jnp.int32)
            for h in range(H):
                pb = plsc.load_gather(p_buf, [kv, jnp.full((16,), h, jnp.int32)])
                for half in range(2):
                    off = (h * 2 + half) * 16
                    xrows[k, pl.ds(off, 16)] = xrows[k, pl.ds(off, 16)] * pb
            return c2

        lax.fori_loop(0, K, scale, 0, unroll=False)

        pltpu.sync_copy(xrows, acc_sh.at[dst_v], add=True)
        pltpu.sync_copy(p_buf, asum_sh.at[dst_v], add=True)
        return carry

    lax.fori_loop(0, NCHUNK, chunk, 0, unroll=False)

    plsc.subcore_barrier()
    row0 = cid * NP + sid * RPT
    pltpu.sync_copy(acc_sh.at[pl.ds(sid * RPT, RPT)],
                    acc_out.at[pl.ds(row0, RPT)])
    pltpu.sync_copy(asum_sh.at[pl.ds(sid * RPT, RPT)],
                    asum_out.at[pl.ds(row0, RPT)])


def _tc_tail_body(acc0_ref, acc1_ref, asum0_ref, asum1_ref, sd_ref, xt_ref,
                  bias_ref, wih0_ref, wih1_ref, wih2_ref, bsum0_ref, bsum1_ref,
                  bsum2_ref, gamma_ref, beta_ref,
                  hout_ref, hnew_ref, cnew_ref):
    sd = sd_ref[...]
    s = sd[:, 0:4]
    d = sd[:, 4:8]
    sa = s + d
    sa = jnp.where(sa > 0.0, sa, sa * 0.2)
    p_self = jnp.exp(sa)
    denom = asum0_ref[:, 0:4] + asum1_ref[:, 0:4] + p_self

    col_h = lax.broadcasted_iota(jnp.int32, (4, HID), 1) // CO
    row_h = lax.broadcasted_iota(jnp.int32, (4, HID), 0)
    sel = (col_h == row_h).astype(jnp.float32)
    p_cols = lax.dot_general(p_self, sel, (((1,), (0,)), ((), ())),
                             preferred_element_type=jnp.float32)
    den_cols = lax.dot_general(denom, sel, (((1,), (0,)), ((), ())),
                               preferred_element_type=jnp.float32)

    numer = acc0_ref[...] + acc1_ref[...] + p_cols * xt_ref[...]
    cur = numer / den_cols + bias_ref[...]

    hs = []
    cs = []
    for wih_ref, bsum_ref in ((wih0_ref, bsum0_ref), (wih1_ref, bsum1_ref),
                              (wih2_ref, bsum2_ref)):
        g = lax.dot_general(cur, wih_ref[...], (((1,), (1,)), ((), ())),
                            preferred_element_type=jnp.float32) + bsum_ref[...]
        gi = jax.nn.sigmoid(g[:, 0:HID])
        gg = jnp.tanh(g[:, 2 * HID:3 * HID])
        go = jax.nn.sigmoid(g[:, 3 * HID:4 * HID])
        c = gi * gg
        h = go * jnp.tanh(c)
        hs.append(h)
        cs.append(c)
        cur = h

    mu = jnp.mean(cur, axis=1, keepdims=True)
    var = jnp.mean((cur - mu) ** 2, axis=1, keepdims=True)
    ln = (cur - mu) * lax.rsqrt(var + 1e-5) * gamma_ref[...] + beta_ref[...]
    hout_ref[...] = ln[None]
    hnew_ref[...] = jnp.stack(hs)
    cnew_ref[...] = jnp.stack(cs)


@jax.jit
def kernel(x, edge_index, edge_attr, W_lin, att_src, att_dst, W_edge,
           att_edge, bias_gat, W_ih0, W_hh0, b_ih0, b_hh0, W_ih1, W_hh1,
           b_ih1, b_hh1, W_ih2, W_hh2, b_ih2, b_hh2, ln_gamma, ln_beta):
    f32 = jnp.float32
    x2d = x.reshape(N, C)

    eye = jnp.eye(H, dtype=f32)
    asd = jnp.concatenate(
        [(att_src[0].astype(f32)[:, :, None] * eye[:, None, :]).reshape(C, H),
         (att_dst[0].astype(f32)[:, :, None] * eye[:, None, :]).reshape(C, H)],
        axis=1)
    ae8 = jnp.concatenate(
        [(att_edge[0].astype(f32)[:, :, None] * eye[:, None, :]).reshape(C, H),
         jnp.zeros((C, H), f32)], axis=1)

    xt, sd = pl.pallas_call(
        _tc_proj_node_body,
        grid=(NGRID,),
        in_specs=[
            pl.BlockSpec((NBLK, C), lambda i: (i, 0)),
            pl.BlockSpec((C, C), lambda i: (0, 0)),
            pl.BlockSpec((C, 8), lambda i: (0, 0)),
        ],
        out_specs=[
            pl.BlockSpec((NBLK, C), lambda i: (i, 0)),
            pl.BlockSpec((NBLK, 8), lambda i: (i, 0)),
        ],
        out_shape=[
            jax.ShapeDtypeStruct((N, C), f32),
            jax.ShapeDtypeStruct((N, 8), f32),
        ],
    )(x2d, W_lin, asd)

    ae = pl.pallas_call(
        _tc_proj_edge_body,
        grid=(EGRID,),
        in_specs=[
            pl.BlockSpec((EBLK, ED), lambda i: (i, 0)),
            pl.BlockSpec((C, ED), lambda i: (0, 0)),
            pl.BlockSpec((C, 8), lambda i: (0, 0)),
        ],
        out_specs=pl.BlockSpec((EBLK, 8), lambda i: (i, 0)),
        out_shape=jax.ShapeDtypeStruct((E, 8), f32),
    )(edge_attr, W_edge, ae8)

    src = edge_index[0]
    dst = edge_index[1]
    z128 = jnp.zeros((NP, HID), f32)
    z8 = jnp.zeros((NP, 8), f32)

    sc_edge = pl.kernel(
        _sc_edge_body,
        out_type=[
            jax.ShapeDtypeStruct((NC * NP, HID), f32),
            jax.ShapeDtypeStruct((NC * NP, 8), f32),
        ],
        mesh=plsc.VectorSubcoreMesh(core_axis_name="c", subcore_axis_name="s"),
        compiler_params=pltpu.CompilerParams(
            needs_layout_passes=False, use_tc_tiling_on_sc=False),
        scratch_types=[
            pltpu.VMEM((K,), jnp.int32),
            pltpu.VMEM((K,), jnp.int32),
            pltpu.VMEM((K, 8), f32),
            pltpu.VMEM((K, 8), f32),
            pltpu.VMEM((K, 8), f32),
            pltpu.VMEM((K, HID), f32),
            pltpu.VMEM((K, 8), f32),
            pltpu.VMEM_SHARED((N, 8), f32),
            pltpu.VMEM_SHARED((NP, HID), f32),
            pltpu.VMEM_SHARED((NP, 8), f32),
            pltpu.SemaphoreType.DMA,
            pltpu.SemaphoreType.DMA,
            pltpu.SemaphoreType.DMA,
        ],
    )
    acc2, asum2 = sc_edge(xt, sd, ae, src, dst, z128, z8)

    bsum0 = (b_ih0 + b_hh0).reshape(1, 4 * HID)
    bsum1 = (b_ih1 + b_hh1).reshape(1, 4 * HID)
    bsum2 = (b_ih2 + b_hh2).reshape(1, 4 * HID)

    h_out, h_new, c_new = pl.pallas_call(
        _tc_tail_body,
        grid=(NGRID,),
        in_specs=[
            pl.BlockSpec((NBLK, HID), lambda i: (i, 0)),
            pl.BlockSpec((NBLK, HID), lambda i: (i, 0)),
            pl.BlockSpec((NBLK, 8), lambda i: (i, 0)),
            pl.BlockSpec((NBLK, 8), lambda i: (i, 0)),
            pl.BlockSpec((NBLK, 8), lambda i: (i, 0)),
            pl.BlockSpec((NBLK, HID), lambda i: (i, 0)),
            pl.BlockSpec((1, HID), lambda i: (0, 0)),
            pl.BlockSpec((4 * HID, HID), lambda i: (0, 0)),
            pl.BlockSpec((4 * HID, HID), lambda i: (0, 0)),
            pl.BlockSpec((4 * HID, HID), lambda i: (0, 0)),
            pl.BlockSpec((1, 4 * HID), lambda i: (0, 0)),
            pl.BlockSpec((1, 4 * HID), lambda i: (0, 0)),
            pl.BlockSpec((1, 4 * HID), lambda i: (0, 0)),
            pl.BlockSpec((1, HID), lambda i: (0, 0)),
            pl.BlockSpec((1, HID), lambda i: (0, 0)),
        ],
        out_specs=[
            pl.BlockSpec((1, NBLK, HID), lambda i: (0, i, 0)),
            pl.BlockSpec((3, NBLK, HID), lambda i: (0, i, 0)),
            pl.BlockSpec((3, NBLK, HID), lambda i: (0, i, 0)),
        ],
        out_shape=[
            jax.ShapeDtypeStruct((1, N, HID), f32),
            jax.ShapeDtypeStruct((3, N, HID), f32),
            jax.ShapeDtypeStruct((3, N, HID), f32),
        ],
    )(acc2[:N], acc2[NP:NP + N], asum2[:N], asum2[NP:NP + N], sd, xt,
      bias_gat.reshape(1, HID), W_ih0, W_ih1, W_ih2, bsum0, bsum1, bsum2,
      ln_gamma.reshape(1, HID), ln_beta.reshape(1, HID))

    return (h_out, h_new, c_new)


# speedup vs baseline: 65.6882x; 2.0614x over previous
import functools

import jax
import jax.numpy as jnp
import numpy as np
from jax import lax
from jax.experimental import pallas as pl
from jax.experimental.pallas import tpu as pltpu
from jax.experimental.pallas import tpu_sc as plsc

N = 10000
E = 320000
C = 128
H = 4
CO = 32
HID = 128
ED = 16

NC = 2
NS = 16
NW = NC * NS
EPW = E // NW
K = 80
NCHUNK = EPW // K
NP = 10240
RPT = NP // NS

NBLK = 2000
NGRID = N // NBLK
EBLK = 2000
EGRID = (E // 16) // EBLK


def _tc_proj_node_body(x_ref, wlin_ref, asd_ref, xt_ref, sd_ref):
    x = x_ref[...]
    xt = lax.dot_general(x, wlin_ref[...], (((1,), (1,)), ((), ())),
                         preferred_element_type=jnp.float32)
    xt_ref[...] = xt
    sd_ref[...] = lax.dot_general(xt, asd_ref[...], (((1,), (0,)), ((), ())),
                                  preferred_element_type=jnp.float32)


def _tc_proj_edge_body(eg_ref, wedge_ref, ae8_ref, ae_ref):
    a_e8 = lax.dot_general(wedge_ref[...], ae8_ref[...], (((0,), (0,)), ((), ())),
                           preferred_element_type=jnp.float32)
    a_rep = jnp.concatenate([a_e8] * 16, axis=0)
    a_rep = jnp.concatenate([a_rep] * 16, axis=1)
    urow = lax.broadcasted_iota(jnp.int32, (16 * ED, HID), 0) // ED
    vcol = lax.broadcasted_iota(jnp.int32, (16 * ED, HID), 1) // 8
    w_big = jnp.where(urow == vcol, a_rep, 0.0)
    ae_ref[...] = lax.dot_general(eg_ref[...], w_big, (((1,), (0,)), ((), ())),
                                  preferred_element_type=jnp.float32)


def _sc_edge_body(xt_hbm, sd_hbm, ae_hbm, src_hbm, dst_hbm, z128_hbm, z8_hbm,
                  acc_out, asum_out,
                  src_v, dst_v, ae_v, srows, drows, xrows, p_buf,
                  sd_sh, acc_sh, asum_sh, sem_x, sem_s, sem_d):
    cid = lax.axis_index("c")
    sid = lax.axis_index("s")
    wid = cid * NS + sid
    spt = N // NS

    pltpu.sync_copy(sd_hbm.at[pl.ds(sid * spt, spt)],
                    sd_sh.at[pl.ds(sid * spt, spt)])
    pltpu.sync_copy(z128_hbm.at[pl.ds(sid * RPT, RPT)],
                    acc_sh.at[pl.ds(sid * RPT, RPT)])
    pltpu.sync_copy(z8_hbm.at[pl.ds(sid * RPT, RPT)],
                    asum_sh.at[pl.ds(sid * RPT, RPT)])
    plsc.subcore_barrier()

    iota16 = lax.iota(jnp.int32, 16)

    def chunk(i, carry):
        base = wid * EPW + i * K
        pltpu.sync_copy(src_hbm.at[pl.ds(base, K)], src_v)
        pltpu.sync_copy(dst_hbm.at[pl.ds(base, K)], dst_v)
        pltpu.sync_copy(ae_hbm.at[pl.ds(base // 16, K // 16)], ae_v)
        cx = pltpu.async_copy(xt_hbm.at[src_v], xrows, sem_x)
        cs = pltpu.async_copy(sd_sh.at[src_v], srows, sem_s)
        cd = pltpu.async_copy(sd_sh.at[dst_v], drows, sem_d)
        cs.wait()
        cd.wait()

        for k16 in range(K // 16):
            rows = iota16 + (k16 * 16)
            k16v = jnp.full((16,), k16, jnp.int32)
            for h in range(H):
                hv = jnp.full((16,), h, jnp.int32)
                hv4 = jnp.full((16,), h + 4, jnp.int32)
                sv = plsc.load_gather(srows, [rows, hv])
                dv = plsc.load_gather(drows, [rows, hv4])
                av = plsc.load_gather(ae_v, [k16v, iota16 * 8 + hv])
                al = sv + dv + av
                al = jnp.where(al > 0.0, al, al * 0.2)
                plsc.store_scatter(p_buf, [rows, hv], jnp.exp(al))

        cx.wait()

        for k16 in range(K // 16):
            rows = iota16 + (k16 * 16)
            p16 = [plsc.load_gather(p_buf, [rows, jnp.full((16,), h, jnp.int32)])
                   for h in range(H)]
            for j in range(16):
                k = k16 * 16 + j
                jv = jnp.full((16,), j, jnp.int32)
                for h in range(H):
                    pb = jnp.take_along_axis(p16[h], jv, axis=0)
                    for half in range(2):
                        off = (h * 2 + half) * 16
                        xrows[k, pl.ds(off, 16)] = xrows[k, pl.ds(off, 16)] * pb

        pltpu.sync_copy(xrows, acc_sh.at[dst_v], add=True)
        pltpu.sync_copy(p_buf, asum_sh.at[dst_v], add=True)
        return carry

    lax.fori_loop(0, NCHUNK, chunk, 0, unroll=False)

    plsc.subcore_barrier()
    row0 = cid * NP + sid * RPT
    pltpu.sync_copy(acc_sh.at[pl.ds(sid * RPT, RPT)],
                    acc_out.at[pl.ds(row0, RPT)])
    pltpu.sync_copy(asum_sh.at[pl.ds(sid * RPT, RPT)],
                    asum_out.at[pl.ds(row0, RPT)])


def _tc_tail_body(acc0_ref, acc1_ref, asum0_ref, asum1_ref, sd_ref, xt_ref,
                  bias_ref, wih0_ref, wih1_ref, wih2_ref, bsum0_ref, bsum1_ref,
                  bsum2_ref, gamma_ref, beta_ref,
                  hout_ref, hnew_ref, cnew_ref):
    sd = sd_ref[...]
    s = sd[:, 0:4]
    d = sd[:, 4:8]
    sa = s + d
    sa = jnp.where(sa > 0.0, sa, sa * 0.2)
    p_self = jnp.exp(sa)
    denom = asum0_ref[0, :, 0:4] + asum1_ref[0, :, 0:4] + p_self

    col_h = lax.broadcasted_iota(jnp.int32, (4, HID), 1) // CO
    row_h = lax.broadcasted_iota(jnp.int32, (4, HID), 0)
    sel = (col_h == row_h).astype(jnp.float32)
    p_cols = lax.dot_general(p_self, sel, (((1,), (0,)), ((), ())),
                             preferred_element_type=jnp.float32)
    den_cols = lax.dot_general(denom, sel, (((1,), (0,)), ((), ())),
                               preferred_element_type=jnp.float32)

    numer = acc0_ref[0] + acc1_ref[0] + p_cols * xt_ref[...]
    cur = numer / den_cols + bias_ref[...]

    hs = []
    cs = []
    for wih_ref, bsum_ref in ((wih0_ref, bsum0_ref), (wih1_ref, bsum1_ref),
                              (wih2_ref, bsum2_ref)):
        g = lax.dot_general(cur, wih_ref[...], (((1,), (1,)), ((), ())),
                            preferred_element_type=jnp.float32) + bsum_ref[...]
        gi = jax.nn.sigmoid(g[:, 0:HID])
        gg = jnp.tanh(g[:, 2 * HID:3 * HID])
        go = jax.nn.sigmoid(g[:, 3 * HID:4 * HID])
        c = gi * gg
        h = go * jnp.tanh(c)
        hs.append(h)
        cs.append(c)
        cur = h

    mu = jnp.mean(cur, axis=1, keepdims=True)
    var = jnp.mean((cur - mu) ** 2, axis=1, keepdims=True)
    ln = (cur - mu) * lax.rsqrt(var + 1e-5) * gamma_ref[...] + beta_ref[...]
    hout_ref[...] = ln[None]
    hnew_ref[...] = jnp.stack(hs)
    cnew_ref[...] = jnp.stack(cs)


@jax.jit
def kernel(x, edge_index, edge_attr, W_lin, att_src, att_dst, W_edge,
           att_edge, bias_gat, W_ih0, W_hh0, b_ih0, b_hh0, W_ih1, W_hh1,
           b_ih1, b_hh1, W_ih2, W_hh2, b_ih2, b_hh2, ln_gamma, ln_beta):
    f32 = jnp.float32
    x2d = x.reshape(N, C)

    eye = jnp.eye(H, dtype=f32)
    asd = jnp.concatenate(
        [(att_src[0].astype(f32)[:, :, None] * eye[:, None, :]).reshape(C, H),
         (att_dst[0].astype(f32)[:, :, None] * eye[:, None, :]).reshape(C, H)],
        axis=1)
    ae8 = jnp.concatenate(
        [(att_edge[0].astype(f32)[:, :, None] * eye[:, None, :]).reshape(C, H),
         jnp.zeros((C, H), f32)], axis=1)

    xt, sd = pl.pallas_call(
        _tc_proj_node_body,
        grid=(NGRID,),
        in_specs=[
            pl.BlockSpec((NBLK, C), lambda i: (i, 0)),
            pl.BlockSpec((C, C), lambda i: (0, 0)),
            pl.BlockSpec((C, 8), lambda i: (0, 0)),
        ],
        out_specs=[
            pl.BlockSpec((NBLK, C), lambda i: (i, 0)),
            pl.BlockSpec((NBLK, 8), lambda i: (i, 0)),
        ],
        out_shape=[
            jax.ShapeDtypeStruct((N, C), f32),
            jax.ShapeDtypeStruct((N, 8), f32),
        ],
    )(x2d, W_lin, asd)

    e_g = edge_attr.reshape(E // 16, 16 * ED)
    ae = pl.pallas_call(
        _tc_proj_edge_body,
        grid=(EGRID,),
        in_specs=[
            pl.BlockSpec((EBLK, 16 * ED), lambda i: (i, 0)),
            pl.BlockSpec((C, ED), lambda i: (0, 0)),
            pl.BlockSpec((C, 8), lambda i: (0, 0)),
        ],
        out_specs=pl.BlockSpec((EBLK, HID), lambda i: (i, 0)),
        out_shape=jax.ShapeDtypeStruct((E // 16, HID), f32),
    )(e_g, W_edge, ae8)

    src = edge_index[0]
    dst = edge_index[1]
    z128 = jnp.zeros((NP, HID), f32)
    z8 = jnp.zeros((NP, 8), f32)

    sc_edge = pl.kernel(
        _sc_edge_body,
        out_type=[
            jax.ShapeDtypeStruct((NC * NP, HID), f32),
            jax.ShapeDtypeStruct((NC * NP, 8), f32),
        ],
        mesh=plsc.VectorSubcoreMesh(core_axis_name="c", subcore_axis_name="s"),
        compiler_params=pltpu.CompilerParams(
            needs_layout_passes=False, use_tc_tiling_on_sc=False),
        scratch_types=[
            pltpu.VMEM((K,), jnp.int32),
            pltpu.VMEM((K,), jnp.int32),
            pltpu.VMEM((K // 16, HID), f32),
            pltpu.VMEM((K, 8), f32),
            pltpu.VMEM((K, 8), f32),
            pltpu.VMEM((K, HID), f32),
            pltpu.VMEM((K, 8), f32),
            pltpu.VMEM_SHARED((N, 8), f32),
            pltpu.VMEM_SHARED((NP, HID), f32),
            pltpu.VMEM_SHARED((NP, 8), f32),
            pltpu.SemaphoreType.DMA,
            pltpu.SemaphoreType.DMA,
            pltpu.SemaphoreType.DMA,
        ],
    )
    acc2, asum2 = sc_edge(xt, sd, ae, src, dst, z128, z8)

    bsum0 = (b_ih0 + b_hh0).reshape(1, 4 * HID)
    bsum1 = (b_ih1 + b_hh1).reshape(1, 4 * HID)
    bsum2 = (b_ih2 + b_hh2).reshape(1, 4 * HID)

    acc3 = acc2.reshape(NC, NP, HID)
    asum3 = asum2.reshape(NC, NP, 8)
    h_out, h_new, c_new = pl.pallas_call(
        _tc_tail_body,
        grid=(NGRID,),
        in_specs=[
            pl.BlockSpec((1, NBLK, HID), lambda i: (0, i, 0)),
            pl.BlockSpec((1, NBLK, HID), lambda i: (1, i, 0)),
            pl.BlockSpec((1, NBLK, 8), lambda i: (0, i, 0)),
            pl.BlockSpec((1, NBLK, 8), lambda i: (1, i, 0)),
            pl.BlockSpec((NBLK, 8), lambda i: (i, 0)),
            pl.BlockSpec((NBLK, HID), lambda i: (i, 0)),
            pl.BlockSpec((1, HID), lambda i: (0, 0)),
            pl.BlockSpec((4 * HID, HID), lambda i: (0, 0)),
            pl.BlockSpec((4 * HID, HID), lambda i: (0, 0)),
            pl.BlockSpec((4 * HID, HID), lambda i: (0, 0)),
            pl.BlockSpec((1, 4 * HID), lambda i: (0, 0)),
            pl.BlockSpec((1, 4 * HID), lambda i: (0, 0)),
            pl.BlockSpec((1, 4 * HID), lambda i: (0, 0)),
            pl.BlockSpec((1, HID), lambda i: (0, 0)),
            pl.BlockSpec((1, HID), lambda i: (0, 0)),
        ],
        out_specs=[
            pl.BlockSpec((1, NBLK, HID), lambda i: (0, i, 0)),
            pl.BlockSpec((3, NBLK, HID), lambda i: (0, i, 0)),
            pl.BlockSpec((3, NBLK, HID), lambda i: (0, i, 0)),
        ],
        out_shape=[
            jax.ShapeDtypeStruct((1, N, HID), f32),
            jax.ShapeDtypeStruct((3, N, HID), f32),
            jax.ShapeDtypeStruct((3, N, HID), f32),
        ],
    )(acc3, acc3, asum3, asum3, sd, xt,
      bias_gat.reshape(1, HID), W_ih0, W_ih1, W_ih2, bsum0, bsum1, bsum2,
      ln_gamma.reshape(1, HID), ln_beta.reshape(1, HID))

    return (h_out, h_new, c_new)


# speedup vs baseline: 70.5152x; 1.0735x over previous
import functools

import jax
import jax.numpy as jnp
import numpy as np
from jax import lax
from jax.experimental import pallas as pl
from jax.experimental.pallas import tpu as pltpu
from jax.experimental.pallas import tpu_sc as plsc

N = 10000
E = 320000
C = 128
H = 4
CO = 32
HID = 128
ED = 16

NC = 2
NS = 16
NW = NC * NS
EPW = E // NW
K = 80
NCHUNK = EPW // K
NP = 10240
RPT = NP // NS

NBLK = 2000
NGRID = N // NBLK
EBLK = 2000
EGRID = (E // 16) // EBLK


def _tc_proj_node_body(x_ref, wlin_ref, asd_ref, xt_ref, sd_ref):
    x = x_ref[...]
    xt = lax.dot_general(x, wlin_ref[...], (((1,), (1,)), ((), ())),
                         preferred_element_type=jnp.float32)
    xt_ref[...] = xt
    sd_ref[...] = lax.dot_general(xt, asd_ref[...], (((1,), (0,)), ((), ())),
                                  preferred_element_type=jnp.float32)


def _tc_proj_edge_body(eg_ref, wedge_ref, ae8_ref, ae_ref):
    a_e8 = lax.dot_general(wedge_ref[...], ae8_ref[...], (((0,), (0,)), ((), ())),
                           preferred_element_type=jnp.float32)
    prod = lax.dot_general(eg_ref[...], a_e8, (((0,), (0,)), ((), ())),
                           preferred_element_type=jnp.float32)
    ae_ref[...] = prod.reshape(EBLK, HID)


def _sc_edge_body(xt_hbm, sd_hbm, ae_hbm, src_hbm, dst_hbm, z128_hbm, z8_hbm,
                  acc_out, asum_out,
                  src_v0, dst_v0, ae_v0, srows0, drows0, xrows0, p_buf0, dst_sc0,
                  src_v1, dst_v1, ae_v1, srows1, drows1, xrows1, p_buf1, dst_sc1,
                  sd_sh, acc_sh, asum_sh, *sems):
    cid = lax.axis_index("c")
    sid = lax.axis_index("s")
    wid = cid * NS + sid
    spt = N // NS

    src_v = [src_v0, src_v1]
    dst_v = [dst_v0, dst_v1]
    ae_v = [ae_v0, ae_v1]
    srows = [srows0, srows1]
    drows = [drows0, drows1]
    xrows = [xrows0, xrows1]
    p_buf = [p_buf0, p_buf1]
    dst_sc = [dst_sc0, dst_sc1]
    sem_ls = sems[0:2]
    sem_ld = sems[2:4]
    sem_la = sems[4:6]
    sem_x = sems[6:8]
    sem_s = sems[8:10]
    sem_d = sems[10:12]
    sem_a = sems[12:14]
    sem_m = sems[14:16]

    pltpu.sync_copy(sd_hbm.at[pl.ds(sid * spt, spt)],
                    sd_sh.at[pl.ds(sid * spt, spt)])
    pltpu.sync_copy(z128_hbm.at[pl.ds(sid * RPT, RPT)],
                    acc_sh.at[pl.ds(sid * RPT, RPT)])
    pltpu.sync_copy(z8_hbm.at[pl.ds(sid * RPT, RPT)],
                    asum_sh.at[pl.ds(sid * RPT, RPT)])
    plsc.subcore_barrier()

    iota16 = lax.iota(jnp.int32, 16)

    def cbase(c):
        return wid * EPW + jnp.minimum(c, NCHUNK - 1) * K

    def issue_linear_srcdst(p, c):
        b = cbase(c)
        pltpu.async_copy(src_hbm.at[pl.ds(b, K)], src_v[p], sem_ls[p])
        pltpu.async_copy(dst_hbm.at[pl.ds(b, K)], dst_v[p], sem_ld[p])

    def issue_linear_ae(p, c):
        b = cbase(c)
        pltpu.async_copy(ae_hbm.at[pl.ds(b // 16, K // 16)], ae_v[p], sem_la[p])

    def wait_linear(p, c):
        b = cbase(c)
        pltpu.make_async_copy(src_hbm.at[pl.ds(b, K)], src_v[p], sem_ls[p]).wait()
        pltpu.make_async_copy(dst_hbm.at[pl.ds(b, K)], dst_v[p], sem_ld[p]).wait()
        pltpu.make_async_copy(ae_hbm.at[pl.ds(b // 16, K // 16)], ae_v[p],
                              sem_la[p]).wait()

    def issue_gathers(p):
        pltpu.async_copy(xt_hbm.at[src_v[p]], xrows[p], sem_x[p])
        pltpu.async_copy(sd_sh.at[src_v[p]], srows[p], sem_s[p])
        pltpu.async_copy(sd_sh.at[dst_v[p]], drows[p], sem_d[p])

    def wait_gathers(p):
        pltpu.make_async_copy(xt_hbm.at[src_v[p]], xrows[p], sem_x[p]).wait()
        pltpu.make_async_copy(sd_sh.at[src_v[p]], srows[p], sem_s[p]).wait()
        pltpu.make_async_copy(sd_sh.at[dst_v[p]], drows[p], sem_d[p]).wait()

    def issue_scatters(p):
        pltpu.async_copy(xrows[p], acc_sh.at[dst_sc[p]], sem_a[p], add=True)
        pltpu.async_copy(p_buf[p], asum_sh.at[dst_sc[p]], sem_m[p], add=True)

    def wait_scatters(p):
        pltpu.make_async_copy(xrows[p], acc_sh.at[dst_sc[p]], sem_a[p]).wait()
        pltpu.make_async_copy(p_buf[p], asum_sh.at[dst_sc[p]], sem_m[p]).wait()

    def snapshot_dst(p, c, maybe_dummy):
        for t in range(K // 16):
            val = dst_v[p][pl.ds(t * 16, 16)]
            if maybe_dummy:
                isdum = jnp.full((16,), c, jnp.int32) >= NCHUNK
                val = jnp.where(isdum, iota16 + (N + t * 16), val)
            dst_sc[p][pl.ds(t * 16, 16)] = val

    def compute_p(p):
        for k16 in range(K // 16):
            rows = iota16 + (k16 * 16)
            k16v = jnp.full((16,), k16, jnp.int32)
            for h in range(H):
                hv = jnp.full((16,), h, jnp.int32)
                hv4 = jnp.full((16,), h + 4, jnp.int32)
                sv = plsc.load_gather(srows[p], [rows, hv])
                dv = plsc.load_gather(drows[p], [rows, hv4])
                av = plsc.load_gather(ae_v[p], [k16v, iota16 * 8 + hv])
                al = sv + dv + av
                al = jnp.where(al > 0.0, al, al * 0.2)
                plsc.store_scatter(p_buf[p], [rows, hv], jnp.exp(al))

    def scale(p):
        for k16 in range(K // 16):
            rows = iota16 + (k16 * 16)
            p16 = [plsc.load_gather(p_buf[p], [rows, jnp.full((16,), h, jnp.int32)])
                   for h in range(H)]
            for j in range(16):
                k = k16 * 16 + j
                jv = jnp.full((16,), j, jnp.int32)
                for h in range(H):
                    pb = jnp.take_along_axis(p16[h], jv, axis=0)
                    for half in range(2):
                        off = (h * 2 + half) * 16
                        xrows[p][k, pl.ds(off, 16)] = (
                            xrows[p][k, pl.ds(off, 16)] * pb)

    def phase(p, c, g, first, maybe_dummy):
        wait_gathers(p)
        snapshot_dst(p, c, maybe_dummy)
        issue_linear_srcdst(p, c + 2)
        compute_p(p)
        issue_linear_ae(p, c + 2)
        q = 1 - p
        if first:
            @pl.when(g > 0)
            def _():
                wait_scatters(q)
        else:
            wait_scatters(q)
        issue_gathers(q)
        scale(p)
        issue_scatters(p)
        wait_linear(p, c + 2)

    issue_linear_srcdst(0, 0)
    issue_linear_ae(0, 0)
    wait_linear(0, 0)
    issue_linear_srcdst(1, 1)
    issue_linear_ae(1, 1)
    wait_linear(1, 1)
    issue_gathers(0)

    def pair(g, carry):
        phase(0, 2 * g, g, True, False)
        phase(1, 2 * g + 1, g, False, True)
        return carry

    lax.fori_loop(0, (NCHUNK + 1) // 2, pair, 0, unroll=False)

    wait_scatters(1)
    wait_gathers(0)

    plsc.subcore_barrier()
    row0 = cid * NP + sid * RPT
    pltpu.sync_copy(acc_sh.at[pl.ds(sid * RPT, RPT)],
                    acc_out.at[pl.ds(row0, RPT)])
    pltpu.sync_copy(asum_sh.at[pl.ds(sid * RPT, RPT)],
                    asum_out.at[pl.ds(row0, RPT)])


def _tc_tail_body(acc0_ref, acc1_ref, asum0_ref, asum1_ref, sd_ref, xt_ref,
                  bias_ref, wih0_ref, wih1_ref, wih2_ref, bsum0_ref, bsum1_ref,
                  bsum2_ref, gamma_ref, beta_ref,
                  hout_ref, hnew_ref, cnew_ref):
    sd = sd_ref[...]
    s = sd[:, 0:4]
    d = sd[:, 4:8]
    sa = s + d
    sa = jnp.where(sa > 0.0, sa, sa * 0.2)
    p_self = jnp.exp(sa)
    denom = asum0_ref[0, :, 0:4] + asum1_ref[0, :, 0:4] + p_self

    col_h = lax.broadcasted_iota(jnp.int32, (4, HID), 1) // CO
    row_h = lax.broadcasted_iota(jnp.int32, (4, HID), 0)
    sel = (col_h == row_h).astype(jnp.float32)
    p_cols = lax.dot_general(p_self, sel, (((1,), (0,)), ((), ())),
                             preferred_element_type=jnp.float32)
    den_cols = lax.dot_general(denom, sel, (((1,), (0,)), ((), ())),
                               preferred_element_type=jnp.float32)

    numer = acc0_ref[0] + acc1_ref[0] + p_cols * xt_ref[...]
    cur = numer / den_cols + bias_ref[...]

    hs = []
    cs = []
    for wih_ref, bsum_ref in ((wih0_ref, bsum0_ref), (wih1_ref, bsum1_ref),
                              (wih2_ref, bsum2_ref)):
        g = lax.dot_general(cur, wih_ref[...], (((1,), (1,)), ((), ())),
                            preferred_element_type=jnp.float32) + bsum_ref[...]
        gi = jax.nn.sigmoid(g[:, 0:HID])
        gg = jnp.tanh(g[:, 2 * HID:3 * HID])
        go = jax.nn.sigmoid(g[:, 3 * HID:4 * HID])
        c = gi * gg
        h = go * jnp.tanh(c)
        hs.append(h)
        cs.append(c)
        cur = h

    mu = jnp.mean(cur, axis=1, keepdims=True)
    var = jnp.mean((cur - mu) ** 2, axis=1, keepdims=True)
    ln = (cur - mu) * lax.rsqrt(var + 1e-5) * gamma_ref[...] + beta_ref[...]
    hout_ref[...] = ln[None]
    hnew_ref[...] = jnp.stack(hs)
    cnew_ref[...] = jnp.stack(cs)


@jax.jit
def kernel(x, edge_index, edge_attr, W_lin, att_src, att_dst, W_edge,
           att_edge, bias_gat, W_ih0, W_hh0, b_ih0, b_hh0, W_ih1, W_hh1,
           b_ih1, b_hh1, W_ih2, W_hh2, b_ih2, b_hh2, ln_gamma, ln_beta):
    f32 = jnp.float32
    x2d = x.reshape(N, C)

    eye = jnp.eye(H, dtype=f32)
    asd = jnp.concatenate(
        [(att_src[0].astype(f32)[:, :, None] * eye[:, None, :]).reshape(C, H),
         (att_dst[0].astype(f32)[:, :, None] * eye[:, None, :]).reshape(C, H)],
        axis=1)
    ae8 = jnp.concatenate(
        [(att_edge[0].astype(f32)[:, :, None] * eye[:, None, :]).reshape(C, H),
         jnp.zeros((C, H), f32)], axis=1)

    xt, sd = pl.pallas_call(
        _tc_proj_node_body,
        grid=(NGRID,),
        in_specs=[
            pl.BlockSpec((NBLK, C), lambda i: (i, 0)),
            pl.BlockSpec((C, C), lambda i: (0, 0)),
            pl.BlockSpec((C, 8), lambda i: (0, 0)),
        ],
        out_specs=[
            pl.BlockSpec((NBLK, C), lambda i: (i, 0)),
            pl.BlockSpec((NBLK, 8), lambda i: (i, 0)),
        ],
        out_shape=[
            jax.ShapeDtypeStruct((N, C), f32),
            jax.ShapeDtypeStruct((N, 8), f32),
        ],
    )(x2d, W_lin, asd)

    e_t3 = edge_attr.T.reshape(ED, E // 16, 16)
    ae = pl.pallas_call(
        _tc_proj_edge_body,
        grid=(EGRID,),
        in_specs=[
            pl.BlockSpec((ED, EBLK, 16), lambda i: (0, i, 0)),
            pl.BlockSpec((C, ED), lambda i: (0, 0)),
            pl.BlockSpec((C, 8), lambda i: (0, 0)),
        ],
        out_specs=pl.BlockSpec((EBLK, HID), lambda i: (i, 0)),
        out_shape=jax.ShapeDtypeStruct((E // 16, HID), f32),
    )(e_t3, W_edge, ae8)

    src = edge_index[0]
    dst = edge_index[1]
    z128 = jnp.zeros((NP, HID), f32)
    z8 = jnp.zeros((NP, 8), f32)

    sc_edge = pl.kernel(
        _sc_edge_body,
        out_type=[
            jax.ShapeDtypeStruct((NC * NP, HID), f32),
            jax.ShapeDtypeStruct((NC * NP, 8), f32),
        ],
        mesh=plsc.VectorSubcoreMesh(core_axis_name="c", subcore_axis_name="s"),
        compiler_params=pltpu.CompilerParams(
            needs_layout_passes=False, use_tc_tiling_on_sc=False),
        scratch_types=(
            [pltpu.VMEM((K,), jnp.int32),
             pltpu.VMEM((K,), jnp.int32),
             pltpu.VMEM((K // 16, HID), f32),
             pltpu.VMEM((K, 8), f32),
             pltpu.VMEM((K, 8), f32),
             pltpu.VMEM((K, HID), f32),
             pltpu.VMEM((K, 8), f32),
             pltpu.VMEM((K,), jnp.int32),
             ] * 2
            + [pltpu.VMEM_SHARED((N, 8), f32),
               pltpu.VMEM_SHARED((NP, HID), f32),
               pltpu.VMEM_SHARED((NP, 8), f32)]
            + [pltpu.SemaphoreType.DMA] * 16
        ),
    )
    acc2, asum2 = sc_edge(xt, sd, ae, src, dst, z128, z8)

    bsum0 = (b_ih0 + b_hh0).reshape(1, 4 * HID)
    bsum1 = (b_ih1 + b_hh1).reshape(1, 4 * HID)
    bsum2 = (b_ih2 + b_hh2).reshape(1, 4 * HID)

    acc3 = acc2.reshape(NC, NP, HID)
    asum3 = asum2.reshape(NC, NP, 8)
    h_out, h_new, c_new = pl.pallas_call(
        _tc_tail_body,
        grid=(NGRID,),
        in_specs=[
            pl.BlockSpec((1, NBLK, HID), lambda i: (0, i, 0)),
            pl.BlockSpec((1, NBLK, HID), lambda i: (1, i, 0)),
            pl.BlockSpec((1, NBLK, 8), lambda i: (0, i, 0)),
            pl.BlockSpec((1, NBLK, 8), lambda i: (1, i, 0)),
            pl.BlockSpec((NBLK, 8), lambda i: (i, 0)),
            pl.BlockSpec((NBLK, HID), lambda i: (i, 0)),
            pl.BlockSpec((1, HID), lambda i: (0, 0)),
            pl.BlockSpec((4 * HID, HID), lambda i: (0, 0)),
            pl.BlockSpec((4 * HID, HID), lambda i: (0, 0)),
            pl.BlockSpec((4 * HID, HID), lambda i: (0, 0)),
            pl.BlockSpec((1, 4 * HID), lambda i: (0, 0)),
            pl.BlockSpec((1, 4 * HID), lambda i: (0, 0)),
            pl.BlockSpec((1, 4 * HID), lambda i: (0, 0)),
            pl.BlockSpec((1, HID), lambda i: (0, 0)),
            pl.BlockSpec((1, HID), lambda i: (0, 0)),
        ],
        out_specs=[
            pl.BlockSpec((1, NBLK, HID), lambda i: (0, i, 0)),
            pl.BlockSpec((3, NBLK, HID), lambda i: (0, i, 0)),
            pl.BlockSpec((3, NBLK, HID), lambda i: (0, i, 0)),
        ],
        out_shape=[
            jax.ShapeDtypeStruct((1, N, HID), f32),
            jax.ShapeDtypeStruct((3, N, HID), f32),
            jax.ShapeDtypeStruct((3, N, HID), f32),
        ],
    )(acc3, acc3, asum3, asum3, sd, xt,
      bias_gat.reshape(1, HID), W_ih0, W_ih1, W_ih2, bsum0, bsum1, bsum2,
      ln_gamma.reshape(1, HID), ln_beta.reshape(1, HID))

    return (h_out, h_new, c_new)


# speedup vs baseline: 94.2186x; 1.3361x over previous
import functools

import jax
import jax.numpy as jnp
import numpy as np
from jax import lax
from jax.experimental import pallas as pl
from jax.experimental.pallas import tpu as pltpu
from jax.experimental.pallas import tpu_sc as plsc

N = 10000
E = 320000
C = 128
H = 4
CO = 32
HID = 128
ED = 16

NC = 2
NS = 16
NW = NC * NS
EPW = E // NW
K = 80
NCHUNK = EPW // K
NP = 10240
RPT = NP // NS

NBLK = 2000
NGRID = N // NBLK
EBLK = 2000
EGRID = (E // 16) // EBLK


def _tc_proj_node_body(x_ref, wlin_ref, asd_ref, xt_ref, sd_ref):
    x = x_ref[...]
    xt = lax.dot_general(x, wlin_ref[...], (((1,), (1,)), ((), ())),
                         preferred_element_type=jnp.float32)
    xt_ref[...] = xt
    sd_ref[...] = lax.dot_general(xt, asd_ref[...], (((1,), (0,)), ((), ())),
                                  preferred_element_type=jnp.float32)


def _tc_proj_edge_body(eg_ref, wedge_ref, ae8_ref, ae_ref):
    a_e8 = lax.dot_general(wedge_ref[...], ae8_ref[...], (((0,), (0,)), ((), ())),
                           preferred_element_type=jnp.float32)
    a_rep = jnp.concatenate([a_e8] * 16, axis=0)
    a_rep = jnp.concatenate([a_rep] * 16, axis=1)
    urow = lax.broadcasted_iota(jnp.int32, (16 * ED, HID), 0) // ED
    vcol = lax.broadcasted_iota(jnp.int32, (16 * ED, HID), 1) // 8
    w_big = jnp.where(urow == vcol, a_rep, 0.0)
    ae_ref[...] = lax.dot_general(eg_ref[...], w_big, (((1,), (0,)), ((), ())),
                                  preferred_element_type=jnp.float32)


def _sc_edge_body(xt_hbm, sd_hbm, ae_hbm, src_hbm, dst_hbm, z128_hbm, z8_hbm,
                  acc_out, asum_out,
                  src_v0, dst_v0, ae_v0, srows0, drows0, xrows0, p_buf0, dst_sc0,
                  src_v1, dst_v1, ae_v1, srows1, drows1, xrows1, p_buf1, dst_sc1,
                  sd_sh, acc_sh, asum_sh, *sems):
    cid = lax.axis_index("c")
    sid = lax.axis_index("s")
    wid = cid * NS + sid
    spt = N // NS

    src_v = [src_v0, src_v1]
    dst_v = [dst_v0, dst_v1]
    ae_v = [ae_v0, ae_v1]
    srows = [srows0, srows1]
    drows = [drows0, drows1]
    xrows = [xrows0, xrows1]
    p_buf = [p_buf0, p_buf1]
    dst_sc = [dst_sc0, dst_sc1]
    sem_ls = sems[0:2]
    sem_ld = sems[2:4]
    sem_la = sems[4:6]
    sem_x = sems[6:8]
    sem_s = sems[8:10]
    sem_d = sems[10:12]
    sem_a = sems[12:14]
    sem_m = sems[14:16]

    pltpu.sync_copy(sd_hbm.at[pl.ds(sid * spt, spt)],
                    sd_sh.at[pl.ds(sid * spt, spt)])
    pltpu.sync_copy(z128_hbm.at[pl.ds(sid * RPT, RPT)],
                    acc_sh.at[pl.ds(sid * RPT, RPT)])
    pltpu.sync_copy(z8_hbm.at[pl.ds(sid * RPT, RPT)],
                    asum_sh.at[pl.ds(sid * RPT, RPT)])
    plsc.subcore_barrier()

    iota16 = lax.iota(jnp.int32, 16)

    def cbase(c):
        return wid * EPW + jnp.minimum(c, NCHUNK - 1) * K

    def issue_linear_srcdst(p, c):
        b = cbase(c)
        pltpu.async_copy(src_hbm.at[pl.ds(b, K)], src_v[p], sem_ls[p])
        pltpu.async_copy(dst_hbm.at[pl.ds(b, K)], dst_v[p], sem_ld[p])

    def issue_linear_ae(p, c):
        b = cbase(c)
        pltpu.async_copy(ae_hbm.at[pl.ds(b // 16, K // 16)], ae_v[p], sem_la[p])

    def wait_linear(p, c):
        b = cbase(c)
        pltpu.make_async_copy(src_hbm.at[pl.ds(b, K)], src_v[p], sem_ls[p]).wait()
        pltpu.make_async_copy(dst_hbm.at[pl.ds(b, K)], dst_v[p], sem_ld[p]).wait()
        pltpu.make_async_copy(ae_hbm.at[pl.ds(b // 16, K // 16)], ae_v[p],
                              sem_la[p]).wait()

    def issue_gathers(p):
        pltpu.async_copy(xt_hbm.at[src_v[p]], xrows[p], sem_x[p])
        pltpu.async_copy(sd_sh.at[src_v[p]], srows[p], sem_s[p])
        pltpu.async_copy(sd_sh.at[dst_v[p]], drows[p], sem_d[p])

    def wait_gathers(p):
        pltpu.make_async_copy(xt_hbm.at[src_v[p]], xrows[p], sem_x[p]).wait()
        pltpu.make_async_copy(sd_sh.at[src_v[p]], srows[p], sem_s[p]).wait()
        pltpu.make_async_copy(sd_sh.at[dst_v[p]], drows[p], sem_d[p]).wait()

    def issue_scatters(p):
        pltpu.async_copy(xrows[p], acc_sh.at[dst_sc[p]], sem_a[p], add=True)
        pltpu.async_copy(p_buf[p], asum_sh.at[dst_sc[p]], sem_m[p], add=True)

    def wait_scatters(p):
        pltpu.make_async_copy(xrows[p], acc_sh.at[dst_sc[p]], sem_a[p]).wait()
        pltpu.make_async_copy(p_buf[p], asum_sh.at[dst_sc[p]], sem_m[p]).wait()

    def snapshot_dst(p, c, maybe_dummy):
        for t in range(K // 16):
            val = dst_v[p][pl.ds(t * 16, 16)]
            if maybe_dummy:
                isdum = jnp.full((16,), c, jnp.int32) >= NCHUNK
                val = jnp.where(isdum, iota16 + (N + t * 16), val)
            dst_sc[p][pl.ds(t * 16, 16)] = val

    def compute_p(p):
        for k16 in range(K // 16):
            rows = iota16 + (k16 * 16)
            k16v = jnp.full((16,), k16, jnp.int32)
            for h in range(H):
                hv = jnp.full((16,), h, jnp.int32)
                hv4 = jnp.full((16,), h + 4, jnp.int32)
                sv = plsc.load_gather(srows[p], [rows, hv])
                dv = plsc.load_gather(drows[p], [rows, hv4])
                av = plsc.load_gather(ae_v[p], [k16v, iota16 * 8 + hv])
                al = sv + dv + av
                al = jnp.where(al > 0.0, al, al * 0.2)
                plsc.store_scatter(p_buf[p], [rows, hv], jnp.exp(al))

    def scale(p):
        for k16 in range(K // 16):
            rows = iota16 + (k16 * 16)
            p16 = [plsc.load_gather(p_buf[p], [rows, jnp.full((16,), h, jnp.int32)])
                   for h in range(H)]
            for j in range(16):
                k = k16 * 16 + j
                jv = jnp.full((16,), j, jnp.int32)
                for h in range(H):
                    pb = jnp.take_along_axis(p16[h], jv, axis=0)
                    for half in range(2):
                        off = (h * 2 + half) * 16
                        xrows[p][k, pl.ds(off, 16)] = (
                            xrows[p][k, pl.ds(off, 16)] * pb)

    def phase(p, c, g, first, maybe_dummy):
        wait_gathers(p)
        snapshot_dst(p, c, maybe_dummy)
        issue_linear_srcdst(p, c + 2)
        compute_p(p)
        issue_linear_ae(p, c + 2)
        q = 1 - p
        if first:
            @pl.when(g > 0)
            def _():
                wait_scatters(q)
        else:
            wait_scatters(q)
        issue_gathers(q)
        scale(p)
        issue_scatters(p)
        wait_linear(p, c + 2)

    issue_linear_srcdst(0, 0)
    issue_linear_ae(0, 0)
    wait_linear(0, 0)
    issue_linear_srcdst(1, 1)
    issue_linear_ae(1, 1)
    wait_linear(1, 1)
    issue_gathers(0)

    def pair(g, carry):
        phase(0, 2 * g, g, True, False)
        phase(1, 2 * g + 1, g, False, True)
        return carry

    lax.fori_loop(0, (NCHUNK + 1) // 2, pair, 0, unroll=False)

    wait_scatters(1)
    wait_gathers(0)

    plsc.subcore_barrier()
    row0 = cid * NP + sid * RPT
    pltpu.sync_copy(acc_sh.at[pl.ds(sid * RPT, RPT)],
                    acc_out.at[pl.ds(row0, RPT)])
    pltpu.sync_copy(asum_sh.at[pl.ds(sid * RPT, RPT)],
                    asum_out.at[pl.ds(row0, RPT)])


def _tc_tail_body(acc0_ref, acc1_ref, asum0_ref, asum1_ref, sd_ref, xt_ref,
                  bias_ref, wih0_ref, wih1_ref, wih2_ref, bsum0_ref, bsum1_ref,
                  bsum2_ref, gamma_ref, beta_ref,
                  hout_ref, hnew_ref, cnew_ref):
    sd = sd_ref[...]
    s = sd[:, 0:4]
    d = sd[:, 4:8]
    sa = s + d
    sa = jnp.where(sa > 0.0, sa, sa * 0.2)
    p_self = jnp.exp(sa)
    denom = asum0_ref[0, :, 0:4] + asum1_ref[0, :, 0:4] + p_self

    col_h = lax.broadcasted_iota(jnp.int32, (4, HID), 1) // CO
    row_h = lax.broadcasted_iota(jnp.int32, (4, HID), 0)
    sel = (col_h == row_h).astype(jnp.float32)
    p_cols = lax.dot_general(p_self, sel, (((1,), (0,)), ((), ())),
                             preferred_element_type=jnp.float32)
    den_cols = lax.dot_general(denom, sel, (((1,), (0,)), ((), ())),
                               preferred_element_type=jnp.float32)

    numer = acc0_ref[0] + acc1_ref[0] + p_cols * xt_ref[...]
    cur = numer / den_cols + bias_ref[...]

    hs = []
    cs = []
    for wih_ref, bsum_ref in ((wih0_ref, bsum0_ref), (wih1_ref, bsum1_ref),
                              (wih2_ref, bsum2_ref)):
        g = lax.dot_general(cur, wih_ref[...], (((1,), (1,)), ((), ())),
                            preferred_element_type=jnp.float32) + bsum_ref[...]
        gi = jax.nn.sigmoid(g[:, 0:HID])
        gg = jnp.tanh(g[:, 2 * HID:3 * HID])
        go = jax.nn.sigmoid(g[:, 3 * HID:4 * HID])
        c = gi * gg
        h = go * jnp.tanh(c)
        hs.append(h)
        cs.append(c)
        cur = h

    mu = jnp.mean(cur, axis=1, keepdims=True)
    var = jnp.mean((cur - mu) ** 2, axis=1, keepdims=True)
    ln = (cur - mu) * lax.rsqrt(var + 1e-5) * gamma_ref[...] + beta_ref[...]
    hout_ref[...] = ln[None]
    hnew_ref[...] = jnp.stack(hs)
    cnew_ref[...] = jnp.stack(cs)


@jax.jit
def kernel(x, edge_index, edge_attr, W_lin, att_src, att_dst, W_edge,
           att_edge, bias_gat, W_ih0, W_hh0, b_ih0, b_hh0, W_ih1, W_hh1,
           b_ih1, b_hh1, W_ih2, W_hh2, b_ih2, b_hh2, ln_gamma, ln_beta):
    f32 = jnp.float32
    x2d = x.reshape(N, C)

    eye = jnp.eye(H, dtype=f32)
    asd = jnp.concatenate(
        [(att_src[0].astype(f32)[:, :, None] * eye[:, None, :]).reshape(C, H),
         (att_dst[0].astype(f32)[:, :, None] * eye[:, None, :]).reshape(C, H)],
        axis=1)
    ae8 = jnp.concatenate(
        [(att_edge[0].astype(f32)[:, :, None] * eye[:, None, :]).reshape(C, H),
         jnp.zeros((C, H), f32)], axis=1)

    xt, sd = pl.pallas_call(
        _tc_proj_node_body,
        grid=(NGRID,),
        in_specs=[
            pl.BlockSpec((NBLK, C), lambda i: (i, 0)),
            pl.BlockSpec((C, C), lambda i: (0, 0)),
            pl.BlockSpec((C, 8), lambda i: (0, 0)),
        ],
        out_specs=[
            pl.BlockSpec((NBLK, C), lambda i: (i, 0)),
            pl.BlockSpec((NBLK, 8), lambda i: (i, 0)),
        ],
        out_shape=[
            jax.ShapeDtypeStruct((N, C), f32),
            jax.ShapeDtypeStruct((N, 8), f32),
        ],
    )(x2d, W_lin, asd)

    e_g = edge_attr.reshape(E // 16, 16 * ED)
    ae = pl.pallas_call(
        _tc_proj_edge_body,
        grid=(EGRID,),
        in_specs=[
            pl.BlockSpec((EBLK, 16 * ED), lambda i: (i, 0)),
            pl.BlockSpec((C, ED), lambda i: (0, 0)),
            pl.BlockSpec((C, 8), lambda i: (0, 0)),
        ],
        out_specs=pl.BlockSpec((EBLK, HID), lambda i: (i, 0)),
        out_shape=jax.ShapeDtypeStruct((E // 16, HID), f32),
    )(e_g, W_edge, ae8)

    src = edge_index[0]
    dst = edge_index[1]
    z128 = jnp.zeros((NP, HID), f32)
    z8 = jnp.zeros((NP, 8), f32)

    sc_edge = pl.kernel(
        _sc_edge_body,
        out_type=[
            jax.ShapeDtypeStruct((NC * NP, HID), f32),
            jax.ShapeDtypeStruct((NC * NP, 8), f32),
        ],
        mesh=plsc.VectorSubcoreMesh(core_axis_name="c", subcore_axis_name="s"),
        compiler_params=pltpu.CompilerParams(
            needs_layout_passes=False, use_tc_tiling_on_sc=False),
        scratch_types=(
            [pltpu.VMEM((K,), jnp.int32),
             pltpu.VMEM((K,), jnp.int32),
             pltpu.VMEM((K // 16, HID), f32),
             pltpu.VMEM((K, 8), f32),
             pltpu.VMEM((K, 8), f32),
             pltpu.VMEM((K, HID), f32),
             pltpu.VMEM((K, 8), f32),
             pltpu.VMEM((K,), jnp.int32),
             ] * 2
            + [pltpu.VMEM_SHARED((N, 8), f32),
               pltpu.VMEM_SHARED((NP, HID), f32),
               pltpu.VMEM_SHARED((NP, 8), f32)]
            + [pltpu.SemaphoreType.DMA] * 16
        ),
    )
    acc2, asum2 = sc_edge(xt, sd, ae, src, dst, z128, z8)

    bsum0 = (b_ih0 + b_hh0).reshape(1, 4 * HID)
    bsum1 = (b_ih1 + b_hh1).reshape(1, 4 * HID)
    bsum2 = (b_ih2 + b_hh2).reshape(1, 4 * HID)

    acc3 = acc2.reshape(NC, NP, HID)
    asum3 = asum2.reshape(NC, NP, 8)
    h_out, h_new, c_new = pl.pallas_call(
        _tc_tail_body,
        grid=(NGRID,),
        in_specs=[
            pl.BlockSpec((1, NBLK, HID), lambda i: (0, i, 0)),
            pl.BlockSpec((1, NBLK, HID), lambda i: (1, i, 0)),
            pl.BlockSpec((1, NBLK, 8), lambda i: (0, i, 0)),
            pl.BlockSpec((1, NBLK, 8), lambda i: (1, i, 0)),
            pl.BlockSpec((NBLK, 8), lambda i: (i, 0)),
            pl.BlockSpec((NBLK, HID), lambda i: (i, 0)),
            pl.BlockSpec((1, HID), lambda i: (0, 0)),
            pl.BlockSpec((4 * HID, HID), lambda i: (0, 0)),
            pl.BlockSpec((4 * HID, HID), lambda i: (0, 0)),
            pl.BlockSpec((4 * HID, HID), lambda i: (0, 0)),
            pl.BlockSpec((1, 4 * HID), lambda i: (0, 0)),
            pl.BlockSpec((1, 4 * HID), lambda i: (0, 0)),
            pl.BlockSpec((1, 4 * HID), lambda i: (0, 0)),
            pl.BlockSpec((1, HID), lambda i: (0, 0)),
            pl.BlockSpec((1, HID), lambda i: (0, 0)),
        ],
        out_specs=[
            pl.BlockSpec((1, NBLK, HID), lambda i: (0, i, 0)),
            pl.BlockSpec((3, NBLK, HID), lambda i: (0, i, 0)),
            pl.BlockSpec((3, NBLK, HID), lambda i: (0, i, 0)),
        ],
        out_shape=[
            jax.ShapeDtypeStruct((1, N, HID), f32),
            jax.ShapeDtypeStruct((3, N, HID), f32),
            jax.ShapeDtypeStruct((3, N, HID), f32),
        ],
    )(acc3, acc3, asum3, asum3, sd, xt,
      bias_gat.reshape(1, HID), W_ih0, W_ih1, W_ih2, bsum0, bsum1, bsum2,
      ln_gamma.reshape(1, HID), ln_beta.reshape(1, HID))

    return (h_out, h_new, c_new)


# speedup vs baseline: 101.2917x; 1.0751x over previous
import functools

import jax
import jax.numpy as jnp
import numpy as np
from jax import lax
from jax.experimental import pallas as pl
from jax.experimental.pallas import tpu as pltpu
from jax.experimental.pallas import tpu_sc as plsc

N = 10000
E = 320000
C = 128
H = 4
CO = 32
HID = 128
ED = 16

NC = 2
NS = 16
NW = NC * NS
EPW = E // NW
K = 80
NCHUNK = EPW // K
NP = 10240
RPT = NP // NS

NBLK = 2000
NGRID = N // NBLK
EBLK = 2000
EGRID = (E // 16) // EBLK


def _tc_proj_node_body(x_ref, wlin_ref, asd_ref, wedge_ref, ae8_ref,
                       xt_ref, sd_ref, a_e8_ref):
    x = x_ref[...]
    xt = lax.dot_general(x, wlin_ref[...], (((1,), (1,)), ((), ())),
                         preferred_element_type=jnp.float32)
    xt_ref[...] = xt
    sd_ref[...] = lax.dot_general(xt, asd_ref[...], (((1,), (0,)), ((), ())),
                                  preferred_element_type=jnp.float32)
    a_e8_ref[...] = lax.dot_general(wedge_ref[...], ae8_ref[...],
                                    (((0,), (0,)), ((), ())),
                                    preferred_element_type=jnp.float32)


def _sc_edge_body(xt_hbm, sd_hbm, eat_hbm, a_hbm, src_hbm, dst_hbm, z128_hbm,
                  z8_hbm, acc_out, asum_out,
                  src_v0, dst_v0, ae_v0, srows0, drows0, xrows0, p_buf0, dst_sc0,
                  src_v1, dst_v1, ae_v1, srows1, drows1, xrows1, p_buf1, dst_sc1,
                  a_v, sd_sh, acc_sh, asum_sh, *sems):
    cid = lax.axis_index("c")
    sid = lax.axis_index("s")
    wid = cid * NS + sid
    spt = N // NS

    src_v = [src_v0, src_v1]
    dst_v = [dst_v0, dst_v1]
    ae_v = [ae_v0, ae_v1]
    srows = [srows0, srows1]
    drows = [drows0, drows1]
    xrows = [xrows0, xrows1]
    p_buf = [p_buf0, p_buf1]
    dst_sc = [dst_sc0, dst_sc1]
    sem_ls = sems[0:2]
    sem_ld = sems[2:4]
    sem_la = sems[4:6]
    sem_x = sems[6:8]
    sem_s = sems[8:10]
    sem_d = sems[10:12]
    sem_a = sems[12:14]
    sem_m = sems[14:16]

    pltpu.sync_copy(a_hbm, a_v)
    pltpu.sync_copy(sd_hbm.at[pl.ds(sid * spt, spt)],
                    sd_sh.at[pl.ds(sid * spt, spt)])
    pltpu.sync_copy(z128_hbm.at[pl.ds(sid * RPT, RPT)],
                    acc_sh.at[pl.ds(sid * RPT, RPT)])
    pltpu.sync_copy(z8_hbm.at[pl.ds(sid * RPT, RPT)],
                    asum_sh.at[pl.ds(sid * RPT, RPT)])
    plsc.subcore_barrier()

    iota16 = lax.iota(jnp.int32, 16)

    def cbase(c):
        return wid * EPW + jnp.minimum(c, NCHUNK - 1) * K

    def issue_linear_srcdst(p, c):
        b = cbase(c)
        pltpu.async_copy(src_hbm.at[pl.ds(b, K)], src_v[p], sem_ls[p])
        pltpu.async_copy(dst_hbm.at[pl.ds(b, K)], dst_v[p], sem_ld[p])

    def issue_linear_ae(p, c):
        b = cbase(c)
        pltpu.async_copy(eat_hbm.at[:, pl.ds(b, K)], ae_v[p], sem_la[p])

    def wait_linear(p, c):
        b = cbase(c)
        pltpu.make_async_copy(src_hbm.at[pl.ds(b, K)], src_v[p], sem_ls[p]).wait()
        pltpu.make_async_copy(dst_hbm.at[pl.ds(b, K)], dst_v[p], sem_ld[p]).wait()
        pltpu.make_async_copy(eat_hbm.at[:, pl.ds(b, K)], ae_v[p],
                              sem_la[p]).wait()

    def issue_gathers(p):
        pltpu.async_copy(xt_hbm.at[src_v[p]], xrows[p], sem_x[p])
        pltpu.async_copy(sd_sh.at[src_v[p]], srows[p], sem_s[p])
        pltpu.async_copy(sd_sh.at[dst_v[p]], drows[p], sem_d[p])

    def wait_gathers(p):
        pltpu.make_async_copy(xt_hbm.at[src_v[p]], xrows[p], sem_x[p]).wait()
        pltpu.make_async_copy(sd_sh.at[src_v[p]], srows[p], sem_s[p]).wait()
        pltpu.make_async_copy(sd_sh.at[dst_v[p]], drows[p], sem_d[p]).wait()

    def issue_scatters(p):
        pltpu.async_copy(xrows[p], acc_sh.at[dst_sc[p]], sem_a[p], add=True)
        pltpu.async_copy(p_buf[p], asum_sh.at[dst_sc[p]], sem_m[p], add=True)

    def wait_scatters(p):
        pltpu.make_async_copy(xrows[p], acc_sh.at[dst_sc[p]], sem_a[p]).wait()
        pltpu.make_async_copy(p_buf[p], asum_sh.at[dst_sc[p]], sem_m[p]).wait()

    def snapshot_dst(p, c, maybe_dummy):
        for t in range(K // 16):
            val = dst_v[p][pl.ds(t * 16, 16)]
            if maybe_dummy:
                isdum = jnp.full((16,), c, jnp.int32) >= NCHUNK
                val = jnp.where(isdum, iota16 + (N + t * 16), val)
            dst_sc[p][pl.ds(t * 16, 16)] = val

    def compute_p(p):
        acol = [plsc.load_gather(a_v, [iota16, jnp.full((16,), h, jnp.int32)])
                for h in range(H)]
        for k16 in range(K // 16):
            rows = iota16 + (k16 * 16)
            ea_d = [ae_v[p][d, pl.ds(k16 * 16, 16)] for d in range(ED)]
            for h in range(H):
                hv = jnp.full((16,), h, jnp.int32)
                hv4 = jnp.full((16,), h + 4, jnp.int32)
                av = ea_d[0] * jnp.take_along_axis(
                    acol[h], jnp.full((16,), 0, jnp.int32), axis=0)
                for d in range(1, ED):
                    av = av + ea_d[d] * jnp.take_along_axis(
                        acol[h], jnp.full((16,), d, jnp.int32), axis=0)
                sv = plsc.load_gather(srows[p], [rows, hv])
                dv = plsc.load_gather(drows[p], [rows, hv4])
                al = sv + dv + av
                al = jnp.where(al > 0.0, al, al * 0.2)
                plsc.store_scatter(p_buf[p], [rows, hv], jnp.exp(al))

    def scale(p):
        for k16 in range(K // 16):
            rows = iota16 + (k16 * 16)
            p16 = [plsc.load_gather(p_buf[p], [rows, jnp.full((16,), h, jnp.int32)])
                   for h in range(H)]
            for j in range(16):
                k = k16 * 16 + j
                jv = jnp.full((16,), j, jnp.int32)
                for h in range(H):
                    pb = jnp.take_along_axis(p16[h], jv, axis=0)
                    for half in range(2):
                        off = (h * 2 + half) * 16
                        xrows[p][k, pl.ds(off, 16)] = (
                            xrows[p][k, pl.ds(off, 16)] * pb)

    def phase(p, c, g, first, maybe_dummy):
        wait_gathers(p)
        snapshot_dst(p, c, maybe_dummy)
        issue_linear_srcdst(p, c + 2)
        compute_p(p)
        issue_linear_ae(p, c + 2)
        q = 1 - p
        if first:
            @pl.when(g > 0)
            def _():
                wait_scatters(q)
        else:
            wait_scatters(q)
        issue_gathers(q)
        scale(p)
        issue_scatters(p)
        wait_linear(p, c + 2)

    issue_linear_srcdst(0, 0)
    issue_linear_ae(0, 0)
    wait_linear(0, 0)
    issue_linear_srcdst(1, 1)
    issue_linear_ae(1, 1)
    wait_linear(1, 1)
    issue_gathers(0)

    def pair(g, carry):
        phase(0, 2 * g, g, True, False)
        phase(1, 2 * g + 1, g, False, True)
        return carry

    lax.fori_loop(0, (NCHUNK + 1) // 2, pair, 0, unroll=False)

    wait_scatters(1)
    wait_gathers(0)

    plsc.subcore_barrier()
    row0 = cid * NP + sid * RPT
    pltpu.sync_copy(acc_sh.at[pl.ds(sid * RPT, RPT)],
                    acc_out.at[pl.ds(row0, RPT)])
    pltpu.sync_copy(asum_sh.at[pl.ds(sid * RPT, RPT)],
                    asum_out.at[pl.ds(row0, RPT)])


def _tc_tail_body(acc0_ref, acc1_ref, asum0_ref, asum1_ref, sd_ref, xt_ref,
                  bias_ref, wih0_ref, wih1_ref, wih2_ref, bsum0_ref, bsum1_ref,
                  bsum2_ref, gamma_ref, beta_ref,
                  hout_ref, hnew_ref, cnew_ref):
    sd = sd_ref[...]
    s = sd[:, 0:4]
    d = sd[:, 4:8]
    sa = s + d
    sa = jnp.where(sa > 0.0, sa, sa * 0.2)
    p_self = jnp.exp(sa)
    denom = asum0_ref[0, :, 0:4] + asum1_ref[0, :, 0:4] + p_self

    col_h = lax.broadcasted_iota(jnp.int32, (4, HID), 1) // CO
    row_h = lax.broadcasted_iota(jnp.int32, (4, HID), 0)
    sel = (col_h == row_h).astype(jnp.float32)
    p_cols = lax.dot_general(p_self, sel, (((1,), (0,)), ((), ())),
                             preferred_element_type=jnp.float32)
    den_cols = lax.dot_general(denom, sel, (((1,), (0,)), ((), ())),
                               preferred_element_type=jnp.float32)

    numer = acc0_ref[0] + acc1_ref[0] + p_cols * xt_ref[...]
    cur = numer / den_cols + bias_ref[...]

    hs = []
    cs = []
    for wih_ref, bsum_ref in ((wih0_ref, bsum0_ref), (wih1_ref, bsum1_ref),
                              (wih2_ref, bsum2_ref)):
        g = lax.dot_general(cur, wih_ref[...], (((1,), (1,)), ((), ())),
                            preferred_element_type=jnp.float32) + bsum_ref[...]
        gi = jax.nn.sigmoid(g[:, 0:HID])
        gg = jnp.tanh(g[:, 2 * HID:3 * HID])
        go = jax.nn.sigmoid(g[:, 3 * HID:4 * HID])
        c = gi * gg
        h = go * jnp.tanh(c)
        hs.append(h)
        cs.append(c)
        cur = h

    mu = jnp.mean(cur, axis=1, keepdims=True)
    var = jnp.mean((cur - mu) ** 2, axis=1, keepdims=True)
    ln = (cur - mu) * lax.rsqrt(var + 1e-5) * gamma_ref[...] + beta_ref[...]
    hout_ref[...] = ln[None]
    hnew_ref[...] = jnp.stack(hs)
    cnew_ref[...] = jnp.stack(cs)


@jax.jit
def kernel(x, edge_index, edge_attr, W_lin, att_src, att_dst, W_edge,
           att_edge, bias_gat, W_ih0, W_hh0, b_ih0, b_hh0, W_ih1, W_hh1,
           b_ih1, b_hh1, W_ih2, W_hh2, b_ih2, b_hh2, ln_gamma, ln_beta):
    f32 = jnp.float32
    x2d = x.reshape(N, C)

    eye = jnp.eye(H, dtype=f32)
    asd = jnp.concatenate(
        [(att_src[0].astype(f32)[:, :, None] * eye[:, None, :]).reshape(C, H),
         (att_dst[0].astype(f32)[:, :, None] * eye[:, None, :]).reshape(C, H)],
        axis=1)
    ae8 = jnp.concatenate(
        [(att_edge[0].astype(f32)[:, :, None] * eye[:, None, :]).reshape(C, H),
         jnp.zeros((C, H), f32)], axis=1)

    xt, sd, a_e8 = pl.pallas_call(
        _tc_proj_node_body,
        grid=(NGRID,),
        in_specs=[
            pl.BlockSpec((NBLK, C), lambda i: (i, 0)),
            pl.BlockSpec((C, C), lambda i: (0, 0)),
            pl.BlockSpec((C, 8), lambda i: (0, 0)),
            pl.BlockSpec((C, ED), lambda i: (0, 0)),
            pl.BlockSpec((C, 8), lambda i: (0, 0)),
        ],
        out_specs=[
            pl.BlockSpec((NBLK, C), lambda i: (i, 0)),
            pl.BlockSpec((NBLK, 8), lambda i: (i, 0)),
            pl.BlockSpec((ED, 8), lambda i: (0, 0)),
        ],
        out_shape=[
            jax.ShapeDtypeStruct((N, C), f32),
            jax.ShapeDtypeStruct((N, 8), f32),
            jax.ShapeDtypeStruct((ED, 8), f32),
        ],
    )(x2d, W_lin, asd, W_edge, ae8)

    ea_t = edge_attr.T

    src = edge_index[0]
    dst = edge_index[1]
    z128 = jnp.zeros((NP, HID), f32)
    z8 = jnp.zeros((NP, 8), f32)

    sc_edge = pl.kernel(
        _sc_edge_body,
        out_type=[
            jax.ShapeDtypeStruct((NC * NP, HID), f32),
            jax.ShapeDtypeStruct((NC * NP, 8), f32),
        ],
        mesh=plsc.VectorSubcoreMesh(core_axis_name="c", subcore_axis_name="s"),
        compiler_params=pltpu.CompilerParams(
            needs_layout_passes=False, use_tc_tiling_on_sc=False),
        scratch_types=(
            [pltpu.VMEM((K,), jnp.int32),
             pltpu.VMEM((K,), jnp.int32),
             pltpu.VMEM((ED, K), f32),
             pltpu.VMEM((K, 8), f32),
             pltpu.VMEM((K, 8), f32),
             pltpu.VMEM((K, HID), f32),
             pltpu.VMEM((K, 8), f32),
             pltpu.VMEM((K,), jnp.int32),
             ] * 2
            + [pltpu.VMEM((ED, 8), f32),
               pltpu.VMEM_SHARED((N, 8), f32),
               pltpu.VMEM_SHARED((NP, HID), f32),
               pltpu.VMEM_SHARED((NP, 8), f32)]
            + [pltpu.SemaphoreType.DMA] * 16
        ),
    )
    acc2, asum2 = sc_edge(xt, sd, ea_t, a_e8, src, dst, z128, z8)

    bsum0 = (b_ih0 + b_hh0).reshape(1, 4 * HID)
    bsum1 = (b_ih1 + b_hh1).reshape(1, 4 * HID)
    bsum2 = (b_ih2 + b_hh2).reshape(1, 4 * HID)

    acc3 = acc2.reshape(NC, NP, HID)
    asum3 = asum2.reshape(NC, NP, 8)
    h_out, h_new, c_new = pl.pallas_call(
        _tc_tail_body,
        grid=(NGRID,),
        in_specs=[
            pl.BlockSpec((1, NBLK, HID), lambda i: (0, i, 0)),
            pl.BlockSpec((1, NBLK, HID), lambda i: (1, i, 0)),
            pl.BlockSpec((1, NBLK, 8), lambda i: (0, i, 0)),
            pl.BlockSpec((1, NBLK, 8), lambda i: (1, i, 0)),
            pl.BlockSpec((NBLK, 8), lambda i: (i, 0)),
            pl.BlockSpec((NBLK, HID), lambda i: (i, 0)),
            pl.BlockSpec((1, HID), lambda i: (0, 0)),
            pl.BlockSpec((4 * HID, HID), lambda i: (0, 0)),
            pl.BlockSpec((4 * HID, HID), lambda i: (0, 0)),
            pl.BlockSpec((4 * HID, HID), lambda i: (0, 0)),
            pl.BlockSpec((1, 4 * HID), lambda i: (0, 0)),
            pl.BlockSpec((1, 4 * HID), lambda i: (0, 0)),
            pl.BlockSpec((1, 4 * HID), lambda i: (0, 0)),
            pl.BlockSpec((1, HID), lambda i: (0, 0)),
            pl.BlockSpec((1, HID), lambda i: (0, 0)),
        ],
        out_specs=[
            pl.BlockSpec((1, NBLK, HID), lambda i: (0, i, 0)),
            pl.BlockSpec((3, NBLK, HID), lambda i: (0, i, 0)),
            pl.BlockSpec((3, NBLK, HID), lambda i: (0, i, 0)),
        ],
        out_shape=[
            jax.ShapeDtypeStruct((1, N, HID), f32),
            jax.ShapeDtypeStruct((3, N, HID), f32),
            jax.ShapeDtypeStruct((3, N, HID), f32),
        ],
    )(acc3, acc3, asum3, asum3, sd, xt,
      bias_gat.reshape(1, HID), W_ih0, W_ih1, W_ih2, bsum0, bsum1, bsum2,
      ln_gamma.reshape(1, HID), ln_beta.reshape(1, HID))

    return (h_out, h_new, c_new)
